# baseline (speedup 1.0000x reference)
import functools

import jax
import jax.numpy as jnp
from jax import lax
from jax.experimental import pallas as pl
from jax.experimental.pallas import tpu as pltpu
from jax.experimental.pallas import tpu_sc as plsc

T = 1024
V = 50258
JW = 1152
VBLK = 2048
NVB = (V + VBLK - 1) // VBLK
ALPHA = 0.2
CHUNK = 128
NCHUNK = JW // CHUNK


def _lse_body(x_ref, lse_ref, m_ref, s_ref):
    v = pl.program_id(0)
    col = lax.broadcasted_iota(jnp.int32, (T, VBLK), 1) + v * VBLK
    x = jnp.where(col < V, x_ref[...], jnp.float32(-jnp.inf))
    bm = jnp.max(x, axis=1, keepdims=True)

    @pl.when(v == 0)
    def _():
        m_ref[...] = bm
        s_ref[...] = jnp.sum(jnp.exp(x - bm), axis=1, keepdims=True)

    @pl.when(v > 0)
    def _():
        m_old = m_ref[...]
        m_new = jnp.maximum(m_old, bm)
        s_ref[...] = s_ref[...] * jnp.exp(m_old - m_new) + jnp.sum(
            jnp.exp(x - m_new), axis=1, keepdims=True)
        m_ref[...] = m_new

    @pl.when(v == NVB - 1)
    def _():
        lse_ref[...] = m_ref[...] + jnp.log(s_ref[...])


def _lse(x2d):
    return pl.pallas_call(
        _lse_body,
        grid=(NVB,),
        in_specs=[pl.BlockSpec((T, VBLK), lambda v: (jnp.int32(0), v))],
        out_specs=pl.BlockSpec(
            (T, 1), lambda v: (jnp.int32(0), jnp.int32(0))),
        out_shape=jax.ShapeDtypeStruct((T, 1), jnp.float32),
        scratch_shapes=[
            pltpu.VMEM((T, 1), jnp.float32),
            pltpu.VMEM((T, 1), jnp.float32),
        ],
        compiler_params=pltpu.CompilerParams(
            dimension_semantics=("arbitrary",)),
    )(x2d)


GW = 128
NROWSG = (T * V) // GW


def _sc_gather(xg, ext_t):
    info = plsc.get_sparse_core_info()
    nc, ns = info.num_cores, info.num_subcores
    nw = nc * ns
    rpw = T // nw
    mesh = plsc.VectorSubcoreMesh(core_axis_name="c", subcore_axis_name="s")

    @functools.partial(
        pl.kernel,
        out_type=jax.ShapeDtypeStruct((T, JW), jnp.float32),
        mesh=mesh,
        scratch_types=[
            pltpu.VMEM((JW,), jnp.int32),
            pltpu.VMEM((JW,), jnp.int32),
            pltpu.VMEM((JW,), jnp.int32),
            pltpu.VMEM((CHUNK, GW), jnp.float32),
            pltpu.VMEM((CHUNK, GW), jnp.float32),
            pltpu.VMEM((JW,), jnp.float32),
            pltpu.SemaphoreType.DMA,
            pltpu.SemaphoreType.DMA,
        ],
        compiler_params=pltpu.CompilerParams(needs_layout_passes=False),
    )
    def gather_kernel(x_hbm, t_hbm, g_hbm, t_v, ridx_v, cidx_v, buf_a,
                      buf_b, row_v, sem_a, sem_b):
        wid = lax.axis_index("s") * nc + lax.axis_index("c")
        pltpu.sync_copy(t_hbm, t_v)
        lane = lax.broadcasted_iota(jnp.int32, (16,), 0)
        bufs = (buf_a, buf_b)
        sems = (sem_a, sem_b)

        @pl.loop(jnp.int32(0), jnp.int32(rpw))
        def row_body(r):
            i = wid * jnp.int32(rpw) + r
            base = i * jnp.int32(V)
            for c in range(JW // 16):
                f = t_v[pl.ds(c * 16, 16)] + base
                ridx_v[pl.ds(c * 16, 16)] = lax.shift_right_logical(
                    f, jnp.int32(7))
                cidx_v[pl.ds(c * 16, 16)] = lax.bitwise_and(f, jnp.int32(127))
            copies = [None] * NCHUNK

            def fire(ch):
                return pltpu.async_copy(
                    x_hbm.at[ridx_v.at[pl.ds(ch * CHUNK, CHUNK)]],
                    bufs[ch % 2],
                    sems[ch % 2],
                )

            copies[0] = fire(0)
            for ch in range(NCHUNK):
                if ch + 1 < NCHUNK:
                    copies[ch + 1] = fire(ch + 1)
                copies[ch].wait()
                buf = bufs[ch % 2]
                for k in range(CHUNK // 16):
                    i0 = lane + jnp.int32(k * 16)
                    i1 = cidx_v[pl.ds(ch * CHUNK + k * 16, 16)]
                    row_v[pl.ds(ch * CHUNK + k * 16, 16)] = plsc.load_gather(
                        buf, [i0, i1])
            pltpu.sync_copy(row_v, g_hbm.at[i])

    return gather_kernel(xg, ext_t)


def _reduce_body(g_ref, lse_ref, tcol_ref, trow_ref, out_ref):
    gg = g_ref[...]
    lse = lse_ref[...]
    tc = tcol_ref[...]
    tr = trow_ref[...]
    ir = lax.broadcasted_iota(jnp.int32, (T, JW), 0)
    jc = lax.broadcasted_iota(jnp.int32, (T, JW), 1)
    eq = tc == tr
    dup = jnp.sum(
        jnp.where(eq & (ir < jc), jnp.int32(1), jnp.int32(0)),
        axis=0, keepdims=True, dtype=jnp.int32)
    firstocc = dup == jnp.int32(0)
    lp = gg - lse
    f = -jnp.log(jnp.maximum(jnp.float32(1.0) - jnp.exp(lp),
                             jnp.float32(1e-5)))
    mask = (jc < ir) & jnp.logical_not(eq) & firstocc & (jc < T)
    mask = mask | (jc == T)
    zero = jnp.float32(0.0)
    custom = jnp.sum(jnp.where(mask, f, zero))
    mle = jnp.sum(lse) - jnp.sum(jnp.where(ir == jc, gg, zero))
    out_ref[...] = jnp.reshape(mle + jnp.float32(ALPHA) * custom, (1, 1))


def _reduce(g, lse, tcol, trow):
    z2 = lambda i: (jnp.int32(0), jnp.int32(0))
    return pl.pallas_call(
        _reduce_body,
        grid=(1,),
        in_specs=[
            pl.BlockSpec((T, JW), z2),
            pl.BlockSpec((T, 1), z2),
            pl.BlockSpec((T, 1), z2),
            pl.BlockSpec((1, JW), z2),
        ],
        out_specs=pl.BlockSpec((1, 1), z2),
        out_shape=jax.ShapeDtypeStruct((1, 1), jnp.float32),
    )(g, lse, tcol, trow)


def kernel(logits, targets):
    x2d = logits.reshape(T, V)
    t = targets.reshape(T).astype(jnp.int32)
    ext_t = jnp.concatenate([t, jnp.full((JW - T,), V - 1, jnp.int32)])
    trow = jnp.concatenate(
        [t, jnp.full((JW - T,), -1, jnp.int32)]).reshape(1, JW)
    tcol = t.reshape(T, 1)

    g = _sc_gather(x2d.reshape(NROWSG, GW), ext_t)
    lse = _lse(x2d)
    out = _reduce(g, lse, tcol, trow)
    return out.reshape(())


# speedup vs baseline: 15.3735x; 15.3735x over previous
import functools

import jax
import jax.numpy as jnp
from jax import lax
from jax.experimental import pallas as pl
from jax.experimental.pallas import tpu as pltpu
from jax.experimental.pallas import tpu_sc as plsc

T = 1024
V = 50258
JW = 1152
VBLK = 2048
NVB = (V + VBLK - 1) // VBLK
ALPHA = 0.2
CHUNK = 128
NCHUNK = JW // CHUNK


def _lse_body(x_ref, lse_ref, m_ref, s_ref):
    v = pl.program_id(0)
    col = lax.broadcasted_iota(jnp.int32, (T, VBLK), 1) + v * VBLK
    x = jnp.where(col < V, x_ref[...], jnp.float32(-jnp.inf))
    bm = jnp.max(x, axis=1, keepdims=True)

    @pl.when(v == 0)
    def _():
        m_ref[...] = bm
        s_ref[...] = jnp.sum(jnp.exp(x - bm), axis=1, keepdims=True)

    @pl.when(v > 0)
    def _():
        m_old = m_ref[...]
        m_new = jnp.maximum(m_old, bm)
        s_ref[...] = s_ref[...] * jnp.exp(m_old - m_new) + jnp.sum(
            jnp.exp(x - m_new), axis=1, keepdims=True)
        m_ref[...] = m_new

    @pl.when(v == NVB - 1)
    def _():
        lse_ref[...] = m_ref[...] + jnp.log(s_ref[...])


def _lse(x2d):
    return pl.pallas_call(
        _lse_body,
        grid=(NVB,),
        in_specs=[pl.BlockSpec((T, VBLK), lambda v: (jnp.int32(0), v))],
        out_specs=pl.BlockSpec(
            (T, 1), lambda v: (jnp.int32(0), jnp.int32(0))),
        out_shape=jax.ShapeDtypeStruct((T, 1), jnp.float32),
        scratch_shapes=[
            pltpu.VMEM((T, 1), jnp.float32),
            pltpu.VMEM((T, 1), jnp.float32),
        ],
        compiler_params=pltpu.CompilerParams(
            dimension_semantics=("arbitrary",)),
    )(x2d)


def _sc_gather(x2d, ext_t):
    info = plsc.get_sparse_core_info()
    nc, ns = info.num_cores, info.num_subcores
    nw = nc * ns
    rpw = T // nw
    half = rpw // 2
    mesh = plsc.VectorSubcoreMesh(core_axis_name="c", subcore_axis_name="s")

    @functools.partial(
        pl.kernel,
        out_type=jax.ShapeDtypeStruct((T, JW), jnp.float32),
        mesh=mesh,
        scratch_types=[
            pltpu.VMEM((JW,), jnp.int32),
            pltpu.VMEM((V,), jnp.float32),
            pltpu.VMEM((V,), jnp.float32),
            pltpu.VMEM((JW,), jnp.float32),
            pltpu.SemaphoreType.DMA,
            pltpu.SemaphoreType.DMA,
        ],
        compiler_params=pltpu.CompilerParams(needs_layout_passes=False),
    )
    def gather_kernel(x_hbm, t_hbm, g_hbm, t_v, row_a, row_b, row_v,
                      sem_a, sem_b):
        wid = lax.axis_index("s") * nc + lax.axis_index("c")
        base = wid * jnp.int32(rpw)
        pltpu.sync_copy(t_hbm, t_v)

        def extract(src, i):
            for c in range(JW // 16):
                idx = t_v[pl.ds(c * 16, 16)]
                row_v[pl.ds(c * 16, 16)] = plsc.load_gather(src, [idx])
            pltpu.sync_copy(row_v, g_hbm.at[i])

        pltpu.async_copy(x_hbm.at[base], row_a, sem_a)

        @pl.loop(jnp.int32(0), jnp.int32(half))
        def pair_body(u):
            i_a = base + jnp.int32(2) * u
            i_b = i_a + jnp.int32(1)
            cp_b = pltpu.async_copy(x_hbm.at[i_b], row_b, sem_b)
            pltpu.make_async_copy(x_hbm.at[i_a], row_a, sem_a).wait()
            extract(row_a, i_a)

            @pl.when(u < jnp.int32(half - 1))
            def _():
                pltpu.async_copy(x_hbm.at[i_a + jnp.int32(2)], row_a, sem_a)

            cp_b.wait()
            extract(row_b, i_b)

    return gather_kernel(x2d, ext_t)


def _reduce_body(g_ref, lse_ref, tcol_ref, trow_ref, out_ref):
    gg = g_ref[...]
    lse = lse_ref[...]
    tc = tcol_ref[...]
    tr = trow_ref[...]
    ir = lax.broadcasted_iota(jnp.int32, (T, JW), 0)
    jc = lax.broadcasted_iota(jnp.int32, (T, JW), 1)
    eq = tc == tr
    dup = jnp.sum(
        jnp.where(eq & (ir < jc), jnp.int32(1), jnp.int32(0)),
        axis=0, keepdims=True, dtype=jnp.int32)
    firstocc = dup == jnp.int32(0)
    lp = gg - lse
    f = -jnp.log(jnp.maximum(jnp.float32(1.0) - jnp.exp(lp),
                             jnp.float32(1e-5)))
    mask = (jc < ir) & jnp.logical_not(eq) & firstocc & (jc < T)
    mask = mask | (jc == T)
    zero = jnp.float32(0.0)
    custom = jnp.sum(jnp.where(mask, f, zero))
    mle = jnp.sum(lse) - jnp.sum(jnp.where(ir == jc, gg, zero))
    out_ref[...] = jnp.reshape(mle + jnp.float32(ALPHA) * custom, (1, 1))


def _reduce(g, lse, tcol, trow):
    z2 = lambda i: (jnp.int32(0), jnp.int32(0))
    return pl.pallas_call(
        _reduce_body,
        grid=(1,),
        in_specs=[
            pl.BlockSpec((T, JW), z2),
            pl.BlockSpec((T, 1), z2),
            pl.BlockSpec((T, 1), z2),
            pl.BlockSpec((1, JW), z2),
        ],
        out_specs=pl.BlockSpec((1, 1), z2),
        out_shape=jax.ShapeDtypeStruct((1, 1), jnp.float32),
    )(g, lse, tcol, trow)


def kernel(logits, targets):
    x2d = logits.reshape(T, V)
    t = targets.reshape(T).astype(jnp.int32)
    ext_t = jnp.concatenate([t, jnp.full((JW - T,), V - 1, jnp.int32)])
    trow = jnp.concatenate(
        [t, jnp.full((JW - T,), -1, jnp.int32)]).reshape(1, JW)
    tcol = t.reshape(T, 1)

    g = _sc_gather(x2d, ext_t)
    lse = _lse(x2d)
    out = _reduce(g, lse, tcol, trow)
    return out.reshape(())
